# baseline (speedup 1.0000x reference)
import functools

import jax
import jax.numpy as jnp
from jax import lax
from jax.experimental import pallas as pl
from jax.experimental.pallas import tpu as pltpu
from jax.experimental.pallas import tpu_sc as plsc

VOCAB = 50265
HIDDEN = 1024
MAX_POS = 2050
PAD_IDX = 1
EPS = 1e-05
B, S = 4, 2048
NTOK = B * S

NC, NS, L = 2, 16, 16
NW = NC * NS
NHALF = 2
HTOK = NTOK // NHALF
TPW = HTOK // NW
CH = 16
NCHUNK = TPW // CH


def _sc_gather_body(row0, ids_hbm, wtab_hbm, ptab_hbm, rows_hbm,
                    ids_row_v, pos_v, buf0, buf1, obuf0, obuf1,
                    sem_g0, sem_g1, sem_o0, sem_o1):
    c = lax.axis_index("c")
    s = lax.axis_index("s")
    row = row0 + c
    chunk_id = s
    row_base = (row - row0) * S
    base = row_base + chunk_id * TPW

    pltpu.sync_copy(ids_hbm.at[pl.ds(row_base, S)], ids_row_v)

    nslice_prefix = chunk_id * (TPW // L)

    def count_body(j, cnt_vec):
        sl = ids_row_v[pl.ds(j * L, L)]
        is_tok = jnp.where(sl != PAD_IDX, 1, 0).astype(jnp.int32)
        flag = jnp.where(j < nslice_prefix, 1, 0).astype(jnp.int32)
        return cnt_vec + is_tok * flag

    cnt_vec = lax.fori_loop(0, S // L, count_body,
                            jnp.zeros((L,), jnp.int32))
    prefix0 = jnp.sum(cnt_vec)

    def pos_body(i, prefix):
        sl = ids_row_v[pl.ds(chunk_id * TPW + i * L, L)]
        m = sl != PAD_IDX
        mi = jnp.where(m, 1, 0).astype(jnp.int32)
        cs = plsc.cumsum(mi)
        pos = jnp.where(m, prefix + cs + PAD_IDX, PAD_IDX)
        pos_v[pl.ds(i * L, L)] = pos
        return prefix + jnp.sum(mi)

    lax.fori_loop(0, TPW // L, pos_body, prefix0)

    def g_copies(k, buf, sem):
        tok0 = chunk_id * TPW + k * CH
        cw = pltpu.make_async_copy(
            wtab_hbm.at[ids_row_v.at[pl.ds(tok0, CH)]],
            buf.at[pl.ds(0, CH)], sem)
        cp = pltpu.make_async_copy(
            ptab_hbm.at[pos_v.at[pl.ds(k * CH, CH)]],
            buf.at[pl.ds(CH, CH)], sem)
        return cw, cp

    def start_g(k, buf, sem):
        for d in g_copies(k, buf, sem):
            d.start()

    def wait_g(k, buf, sem):
        for d in g_copies(k, buf, sem):
            d.wait()

    def pack_chunk(src, dst):
        def t_body(t, carry):
            def j_body(j, carry2):
                a = src[t, pl.ds(j * L, L)]
                b = src[CH + t, pl.ds(j * L, L)]
                packed = plsc.pack(a, b, format=plsc.PackFormat.INTERLEAVED)
                dst[t, pl.ds(j * L, L)] = plsc.bitcast(packed, jnp.int32)
                return carry2
            return lax.fori_loop(0, HIDDEN // L, j_body, carry)
        lax.fori_loop(0, CH, t_body, 0)

    def o_copy(k, buf, sem):
        out0 = base + k * CH
        return pltpu.make_async_copy(buf, rows_hbm.at[pl.ds(out0, CH)], sem)

    start_g(0, buf0, sem_g0)

    def pipe_body(g, carry):
        k0 = 2 * g
        k1 = 2 * g + 1

        start_g(k1, buf1, sem_g1)
        wait_g(k0, buf0, sem_g0)
        pack_chunk(buf0, obuf0)

        @pl.when(g > 0)
        def _():
            o_copy(k1 - 2, obuf1, sem_o1).wait()

        o_copy(k0, obuf0, sem_o0).start()

        @pl.when(g < NCHUNK // 2 - 1)
        def _():
            start_g(k0 + 2, buf0, sem_g0)

        wait_g(k1, buf1, sem_g1)
        pack_chunk(buf1, obuf1)
        o_copy(k1, obuf1, sem_o1).start()
        o_copy(k0, obuf0, sem_o0).wait()
        return carry

    lax.fori_loop(0, NCHUNK // 2, pipe_body, 0)
    o_copy(NCHUNK - 1, obuf1, sem_o1).wait()


def _sc_gather(row0, ids_half, word_embeddings, position_embeddings):
    mesh = plsc.VectorSubcoreMesh(core_axis_name="c", subcore_axis_name="s",
                                  num_cores=NC, num_subcores=NS)
    f = pl.kernel(
        functools.partial(_sc_gather_body, row0),
        out_type=jax.ShapeDtypeStruct((HTOK, HIDDEN), jnp.int32),
        mesh=mesh,
        compiler_params=pltpu.CompilerParams(needs_layout_passes=False),
        scratch_types=[
            pltpu.VMEM((S,), jnp.int32),
            pltpu.VMEM((TPW,), jnp.int32),
            pltpu.VMEM((2 * CH, HIDDEN), jnp.float32),
            pltpu.VMEM((2 * CH, HIDDEN), jnp.float32),
            pltpu.VMEM((CH, HIDDEN), jnp.int32),
            pltpu.VMEM((CH, HIDDEN), jnp.int32),
            pltpu.SemaphoreType.DMA,
            pltpu.SemaphoreType.DMA,
            pltpu.SemaphoreType.DMA,
            pltpu.SemaphoreType.DMA,
        ],
    )
    return f(ids_half, word_embeddings, position_embeddings)


def _ln_body(r_ref, g_ref, b_ref, o_ref):
    r = r_ref[...]
    lo = lax.bitcast_convert_type(lax.shift_left(r, 16), jnp.float32)
    hi = lax.bitcast_convert_type(
        lax.bitwise_and(r, jnp.int32(-65536)), jnp.float32)
    e = lo + hi
    mean = jnp.mean(e, axis=-1, keepdims=True)
    d = e - mean
    var = jnp.mean(d * d, axis=-1, keepdims=True)
    o_ref[...] = d * lax.rsqrt(var + EPS) * g_ref[...] + b_ref[...]


BLK = 256


def _tc_layernorm(half, acc, rows, ln_gamma, ln_beta):
    grid = (HTOK // BLK,)
    blk0 = half * (HTOK // BLK)
    dense_specs = [
        pl.BlockSpec((BLK, HIDDEN), lambda i: (i, 0)),
        pl.BlockSpec((1, HIDDEN), lambda i: (0, 0)),
        pl.BlockSpec((1, HIDDEN), lambda i: (0, 0)),
    ]
    if half == 0:
        body, in_specs, aliases, args = _ln_body, dense_specs, {}, ()
    else:
        def body(_, r_ref, g_ref, b_ref, o_ref):
            _ln_body(r_ref, g_ref, b_ref, o_ref)
        in_specs = [pl.BlockSpec(memory_space=pl.ANY)] + dense_specs
        aliases = {0: 0}
        args = (acc,)
    return pl.pallas_call(
        body,
        grid=grid,
        in_specs=in_specs,
        out_specs=pl.BlockSpec((BLK, HIDDEN),
                               lambda i: (i + blk0, 0)),
        out_shape=jax.ShapeDtypeStruct((NTOK, HIDDEN), jnp.float32),
        input_output_aliases=aliases,
    )(*args, rows, ln_gamma.reshape(1, HIDDEN), ln_beta.reshape(1, HIDDEN))


def kernel(input_ids, word_embeddings, position_embeddings, ln_gamma, ln_beta):
    ids_flat = input_ids.reshape(NTOK).astype(jnp.int32)
    g2 = ln_gamma.reshape(1, HIDDEN)
    b2 = ln_beta.reshape(1, HIDDEN)
    acc = None
    for h in range(NHALF):
        ids_half = lax.slice(ids_flat, (h * HTOK,), ((h + 1) * HTOK,))
        rows = _sc_gather(h * (B // NHALF), ids_half,
                          word_embeddings, position_embeddings)
        acc = _tc_layernorm(h, acc, rows, g2, b2)
    return acc.reshape(B, S, HIDDEN)


# speedup vs baseline: 1.3009x; 1.3009x over previous
import functools

import jax
import jax.numpy as jnp
from jax import lax
from jax.experimental import pallas as pl
from jax.experimental.pallas import tpu as pltpu
from jax.experimental.pallas import tpu_sc as plsc

VOCAB = 50265
HIDDEN = 1024
MAX_POS = 2050
PAD_IDX = 1
EPS = 1e-05
B, S = 4, 2048
NTOK = B * S

NC, NS, L = 2, 16, 16
NW = NC * NS
NHALF = 2
HTOK = NTOK // NHALF
TPW = HTOK // NW
CH = 16
NCHUNK = TPW // CH


def _sc_gather_body(row0, ids_hbm, wtab_hbm, ptab_hbm, rows_hbm,
                    ids_row_v, pos_v, buf0, buf1, obuf0, obuf1,
                    sem_g0, sem_g1, sem_o0, sem_o1):
    c = lax.axis_index("c")
    s = lax.axis_index("s")
    row = row0 + c
    chunk_id = s
    row_base = (row - row0) * S
    base = row_base + chunk_id * TPW

    pltpu.sync_copy(ids_hbm.at[pl.ds(row_base, S)], ids_row_v)

    nslice_prefix = chunk_id * (TPW // L)

    def count_body(j, cnt_vec):
        sl = ids_row_v[pl.ds(j * L, L)]
        is_tok = jnp.where(sl != PAD_IDX, 1, 0).astype(jnp.int32)
        flag = jnp.where(j < nslice_prefix, 1, 0).astype(jnp.int32)
        return cnt_vec + is_tok * flag

    cnt_vec = lax.fori_loop(0, S // L, count_body,
                            jnp.zeros((L,), jnp.int32))
    prefix0 = jnp.sum(cnt_vec)

    def pos_body(i, prefix):
        sl = ids_row_v[pl.ds(chunk_id * TPW + i * L, L)]
        m = sl != PAD_IDX
        mi = jnp.where(m, 1, 0).astype(jnp.int32)
        cs = plsc.cumsum(mi)
        pos = jnp.where(m, prefix + cs + PAD_IDX, PAD_IDX)
        pos_v[pl.ds(i * L, L)] = pos
        return prefix + jnp.sum(mi)

    lax.fori_loop(0, TPW // L, pos_body, prefix0)

    def g_copies(k, buf, sem):
        tok0 = chunk_id * TPW + k * CH
        cw = pltpu.make_async_copy(
            wtab_hbm.at[ids_row_v.at[pl.ds(tok0, CH)]],
            buf.at[pl.ds(0, CH)], sem)
        cp = pltpu.make_async_copy(
            ptab_hbm.at[pos_v.at[pl.ds(k * CH, CH)]],
            buf.at[pl.ds(CH, CH)], sem)
        return cw, cp

    def start_g(k, buf, sem):
        for d in g_copies(k, buf, sem):
            d.start()

    def wait_g(k, buf, sem):
        for d in g_copies(k, buf, sem):
            d.wait()

    def pack_chunk(src, dst):
        def t_body(t, carry):
            @plsc.parallel_loop(0, HIDDEN // L, unroll=8)
            def _(j):
                a = src[t, pl.ds(j * L, L)]
                b = src[CH + t, pl.ds(j * L, L)]
                packed = plsc.pack(a, b, format=plsc.PackFormat.INTERLEAVED)
                dst[t, pl.ds(j * L, L)] = plsc.bitcast(packed, jnp.int32)
            return carry
        lax.fori_loop(0, CH, t_body, 0)

    def o_copy(k, buf, sem):
        out0 = base + k * CH
        return pltpu.make_async_copy(buf, rows_hbm.at[pl.ds(out0, CH)], sem)

    start_g(0, buf0, sem_g0)

    def pipe_body(g, carry):
        k0 = 2 * g
        k1 = 2 * g + 1

        start_g(k1, buf1, sem_g1)
        wait_g(k0, buf0, sem_g0)
        pack_chunk(buf0, obuf0)

        @pl.when(g > 0)
        def _():
            o_copy(k1 - 2, obuf1, sem_o1).wait()

        o_copy(k0, obuf0, sem_o0).start()

        @pl.when(g < NCHUNK // 2 - 1)
        def _():
            start_g(k0 + 2, buf0, sem_g0)

        wait_g(k1, buf1, sem_g1)
        pack_chunk(buf1, obuf1)
        o_copy(k1, obuf1, sem_o1).start()
        o_copy(k0, obuf0, sem_o0).wait()
        return carry

    lax.fori_loop(0, NCHUNK // 2, pipe_body, 0)
    o_copy(NCHUNK - 1, obuf1, sem_o1).wait()


def _sc_gather(row0, ids_half, word_embeddings, position_embeddings):
    mesh = plsc.VectorSubcoreMesh(core_axis_name="c", subcore_axis_name="s",
                                  num_cores=NC, num_subcores=NS)
    f = pl.kernel(
        functools.partial(_sc_gather_body, row0),
        out_type=jax.ShapeDtypeStruct((HTOK, HIDDEN), jnp.int32),
        mesh=mesh,
        compiler_params=pltpu.CompilerParams(needs_layout_passes=False),
        scratch_types=[
            pltpu.VMEM((S,), jnp.int32),
            pltpu.VMEM((TPW,), jnp.int32),
            pltpu.VMEM((2 * CH, HIDDEN), jnp.float32),
            pltpu.VMEM((2 * CH, HIDDEN), jnp.float32),
            pltpu.VMEM((CH, HIDDEN), jnp.int32),
            pltpu.VMEM((CH, HIDDEN), jnp.int32),
            pltpu.SemaphoreType.DMA,
            pltpu.SemaphoreType.DMA,
            pltpu.SemaphoreType.DMA,
            pltpu.SemaphoreType.DMA,
        ],
    )
    return f(ids_half, word_embeddings, position_embeddings)


def _ln_body(r_ref, g_ref, b_ref, o_ref):
    r = r_ref[...]
    lo = lax.bitcast_convert_type(lax.shift_left(r, 16), jnp.float32)
    hi = lax.bitcast_convert_type(
        lax.bitwise_and(r, jnp.int32(-65536)), jnp.float32)
    e = lo + hi
    mean = jnp.mean(e, axis=-1, keepdims=True)
    d = e - mean
    var = jnp.mean(d * d, axis=-1, keepdims=True)
    o_ref[...] = d * lax.rsqrt(var + EPS) * g_ref[...] + b_ref[...]


BLK = 256


def _tc_layernorm(half, acc, rows, ln_gamma, ln_beta):
    grid = (HTOK // BLK,)
    blk0 = half * (HTOK // BLK)
    dense_specs = [
        pl.BlockSpec((BLK, HIDDEN), lambda i: (i, 0)),
        pl.BlockSpec((1, HIDDEN), lambda i: (0, 0)),
        pl.BlockSpec((1, HIDDEN), lambda i: (0, 0)),
    ]
    if half == 0:
        body, in_specs, aliases, args = _ln_body, dense_specs, {}, ()
    else:
        def body(_, r_ref, g_ref, b_ref, o_ref):
            _ln_body(r_ref, g_ref, b_ref, o_ref)
        in_specs = [pl.BlockSpec(memory_space=pl.ANY)] + dense_specs
        aliases = {0: 0}
        args = (acc,)
    return pl.pallas_call(
        body,
        grid=grid,
        in_specs=in_specs,
        out_specs=pl.BlockSpec((BLK, HIDDEN),
                               lambda i: (i + blk0, 0)),
        out_shape=jax.ShapeDtypeStruct((NTOK, HIDDEN), jnp.float32),
        input_output_aliases=aliases,
    )(*args, rows, ln_gamma.reshape(1, HIDDEN), ln_beta.reshape(1, HIDDEN))


def kernel(input_ids, word_embeddings, position_embeddings, ln_gamma, ln_beta):
    ids_flat = input_ids.reshape(NTOK).astype(jnp.int32)
    g2 = ln_gamma.reshape(1, HIDDEN)
    b2 = ln_beta.reshape(1, HIDDEN)
    acc = None
    for h in range(NHALF):
        ids_half = lax.slice(ids_flat, (h * HTOK,), ((h + 1) * HTOK,))
        rows = _sc_gather(h * (B // NHALF), ids_half,
                          word_embeddings, position_embeddings)
        acc = _tc_layernorm(h, acc, rows, g2, b2)
    return acc.reshape(B, S, HIDDEN)


# speedup vs baseline: 1.3858x; 1.0652x over previous
import functools

import jax
import jax.numpy as jnp
from jax import lax
from jax.experimental import pallas as pl
from jax.experimental.pallas import tpu as pltpu
from jax.experimental.pallas import tpu_sc as plsc

VOCAB = 50265
HIDDEN = 1024
MAX_POS = 2050
PAD_IDX = 1
EPS = 1e-05
B, S = 4, 2048
NTOK = B * S

NC, NS, L = 2, 16, 16
NW = NC * NS
NHALF = 2
HTOK = NTOK // NHALF
TPW = HTOK // NW
CH = 16
NCHUNK = TPW // CH


def _sc_gather_body(row0, ids_hbm, wtab_hbm, ptab_hbm, rows_hbm,
                    ids_row_v, pos_v, buf0, buf1, obuf0, obuf1,
                    sem_g0, sem_g1, sem_o0, sem_o1):
    c = lax.axis_index("c")
    s = lax.axis_index("s")
    row = row0 + c
    chunk_id = s
    row_base = (row - row0) * S
    base = row_base + chunk_id * TPW

    pltpu.sync_copy(ids_hbm.at[pl.ds(row_base, S)], ids_row_v)

    nslice_prefix = chunk_id * (TPW // L)

    def count_body(j, cnt_vec):
        sl = ids_row_v[pl.ds(j * L, L)]
        is_tok = jnp.where(sl != PAD_IDX, 1, 0).astype(jnp.int32)
        flag = jnp.where(j < nslice_prefix, 1, 0).astype(jnp.int32)
        return cnt_vec + is_tok * flag

    cnt_vec = lax.fori_loop(0, S // L, count_body,
                            jnp.zeros((L,), jnp.int32))
    prefix0 = jnp.sum(cnt_vec)

    def pos_body(i, prefix):
        sl = ids_row_v[pl.ds(chunk_id * TPW + i * L, L)]
        m = sl != PAD_IDX
        mi = jnp.where(m, 1, 0).astype(jnp.int32)
        cs = plsc.cumsum(mi)
        pos = jnp.where(m, prefix + cs + PAD_IDX, PAD_IDX)
        pos_v[pl.ds(i * L, L)] = pos
        return prefix + jnp.sum(mi)

    lax.fori_loop(0, TPW // L, pos_body, prefix0)

    def g_copies(k, buf, sem):
        tok0 = chunk_id * TPW + k * CH
        cw = pltpu.make_async_copy(
            wtab_hbm.at[ids_row_v.at[pl.ds(tok0, CH)]],
            buf.at[pl.ds(0, CH)], sem)
        cp = pltpu.make_async_copy(
            ptab_hbm.at[pos_v.at[pl.ds(k * CH, CH)]],
            buf.at[pl.ds(CH, CH)], sem)
        return cw, cp

    def start_g(k, buf, sem):
        for d in g_copies(k, buf, sem):
            d.start()

    def wait_g(k, buf, sem):
        for d in g_copies(k, buf, sem):
            d.wait()

    def pack_chunk(src, dst):
        def t_body(t, carry):
            @plsc.parallel_loop(0, HIDDEN // L, unroll=8)
            def _(j):
                a = src[t, pl.ds(j * L, L)]
                b = src[CH + t, pl.ds(j * L, L)]
                packed = plsc.pack(a, b, format=plsc.PackFormat.INTERLEAVED)
                dst[t, pl.ds(j * L, L)] = plsc.bitcast(packed, jnp.int32)
            return carry
        lax.fori_loop(0, CH, t_body, 0)

    def o_copy(k, buf, sem):
        out0 = base + k * CH
        return pltpu.make_async_copy(buf, rows_hbm.at[pl.ds(out0, CH)], sem)

    start_g(0, buf0, sem_g0)

    def pipe_body(g, carry):
        k0 = 2 * g
        k1 = 2 * g + 1

        start_g(k1, buf1, sem_g1)
        wait_g(k0, buf0, sem_g0)
        pack_chunk(buf0, obuf0)

        @pl.when(g > 0)
        def _():
            o_copy(k1 - 2, obuf1, sem_o1).wait()

        o_copy(k0, obuf0, sem_o0).start()

        @pl.when(g < NCHUNK // 2 - 1)
        def _():
            start_g(k0 + 2, buf0, sem_g0)

        wait_g(k1, buf1, sem_g1)
        pack_chunk(buf1, obuf1)
        o_copy(k1, obuf1, sem_o1).start()
        o_copy(k0, obuf0, sem_o0).wait()
        return carry

    lax.fori_loop(0, NCHUNK // 2, pipe_body, 0)
    o_copy(NCHUNK - 1, obuf1, sem_o1).wait()


def _sc_gather(row0, ids_half, word_embeddings, position_embeddings):
    mesh = plsc.VectorSubcoreMesh(core_axis_name="c", subcore_axis_name="s",
                                  num_cores=NC, num_subcores=NS)
    f = pl.kernel(
        functools.partial(_sc_gather_body, row0),
        out_type=jax.ShapeDtypeStruct((HTOK, HIDDEN), jnp.int32),
        mesh=mesh,
        compiler_params=pltpu.CompilerParams(needs_layout_passes=False),
        scratch_types=[
            pltpu.VMEM((S,), jnp.int32),
            pltpu.VMEM((TPW,), jnp.int32),
            pltpu.VMEM((2 * CH, HIDDEN), jnp.float32),
            pltpu.VMEM((2 * CH, HIDDEN), jnp.float32),
            pltpu.VMEM((CH, HIDDEN), jnp.int32),
            pltpu.VMEM((CH, HIDDEN), jnp.int32),
            pltpu.SemaphoreType.DMA,
            pltpu.SemaphoreType.DMA,
            pltpu.SemaphoreType.DMA,
            pltpu.SemaphoreType.DMA,
        ],
    )
    return f(ids_half, word_embeddings, position_embeddings)


def _ln_body(r_ref, g_ref, b_ref, o_ref):
    r = r_ref[...]
    lo = lax.bitcast_convert_type(lax.shift_left(r, 16), jnp.float32)
    hi = lax.bitcast_convert_type(
        lax.bitwise_and(r, jnp.int32(-65536)), jnp.float32)
    e = lo + hi
    mean = jnp.mean(e, axis=-1, keepdims=True)
    d = e - mean
    var = jnp.mean(d * d, axis=-1, keepdims=True)
    o_ref[...] = d * lax.rsqrt(var + EPS) * g_ref[...] + b_ref[...]


BLK = 512


def _tc_layernorm(half, acc, rows, ln_gamma, ln_beta):
    grid = (HTOK // BLK,)
    blk0 = half * (HTOK // BLK)
    dense_specs = [
        pl.BlockSpec((BLK, HIDDEN), lambda i: (i, 0)),
        pl.BlockSpec((1, HIDDEN), lambda i: (0, 0)),
        pl.BlockSpec((1, HIDDEN), lambda i: (0, 0)),
    ]
    if half == 0:
        body, in_specs, aliases, args = _ln_body, dense_specs, {}, ()
    else:
        def body(_, r_ref, g_ref, b_ref, o_ref):
            _ln_body(r_ref, g_ref, b_ref, o_ref)
        in_specs = [pl.BlockSpec(memory_space=pl.ANY)] + dense_specs
        aliases = {0: 0}
        args = (acc,)
    return pl.pallas_call(
        body,
        grid=grid,
        in_specs=in_specs,
        out_specs=pl.BlockSpec((BLK, HIDDEN),
                               lambda i: (i + blk0, 0)),
        out_shape=jax.ShapeDtypeStruct((NTOK, HIDDEN), jnp.float32),
        input_output_aliases=aliases,
    )(*args, rows, ln_gamma.reshape(1, HIDDEN), ln_beta.reshape(1, HIDDEN))


def kernel(input_ids, word_embeddings, position_embeddings, ln_gamma, ln_beta):
    ids_flat = input_ids.reshape(NTOK).astype(jnp.int32)
    g2 = ln_gamma.reshape(1, HIDDEN)
    b2 = ln_beta.reshape(1, HIDDEN)
    acc = None
    for h in range(NHALF):
        ids_half = lax.slice(ids_flat, (h * HTOK,), ((h + 1) * HTOK,))
        rows = _sc_gather(h * (B // NHALF), ids_half,
                          word_embeddings, position_embeddings)
        acc = _tc_layernorm(h, acc, rows, g2, b2)
    return acc.reshape(B, S, HIDDEN)
